# baseline (speedup 1.0000x reference)
import functools

import jax
import jax.numpy as jnp
from jax import lax
from jax.experimental import pallas as pl
from jax.experimental.pallas import tpu as pltpu
from jax.experimental.pallas import tpu_sc as plsc

B = 16384
F = 64
NC = 2
NS = 16
NW = NC * NS
BPW = B // NW
BLK = 128
NBUF = 4
PF = NBUF - 1
NBUFI = 7
PFI = NBUFI - 1

_mesh = plsc.VectorSubcoreMesh(core_axis_name="c", subcore_axis_name="s")


@functools.partial(
    pl.kernel,
    mesh=_mesh,
    out_type=jax.ShapeDtypeStruct((NW, BPW), jnp.float32),
    scratch_types=[
        pltpu.VMEM((2, BPW), jnp.int32),
        pltpu.SMEM((BPW,), jnp.int32),
        pltpu.SMEM((BPW,), jnp.int32),
        pltpu.VMEM((NBUF, F, BLK), jnp.float32),
        pltpu.VMEM((NBUFI, F, BLK), jnp.float32),
        pltpu.VMEM((F, BPW), jnp.float32),
        pltpu.VMEM((BPW,), jnp.float32),
        pltpu.SemaphoreType.DMA,
        pltpu.SemaphoreType.DMA,
    ],
    compiler_params=pltpu.CompilerParams(needs_layout_passes=False),
)
def _cf_kernel(user_hbm, item_hbm, utT_hbm, itT_hbm, out_hbm,
               idxstage_v, uidx_s, iidx_s, ublk_v, iblk_v,
               prodT_v, out_v, usem, isem):
    wid = lax.axis_index("s") * NC + lax.axis_index("c")

    pltpu.sync_copy(user_hbm.at[wid], idxstage_v.at[0])
    pltpu.sync_copy(item_hbm.at[wid], idxstage_v.at[1])

    def smem_fill(g, carry):
        u16 = idxstage_v[0, pl.ds(g * 16, 16)]
        i16 = idxstage_v[1, pl.ds(g * 16, 16)]
        for l in range(16):
            uidx_s[g * 16 + l] = u16[l]
            iidx_s[g * 16 + l] = i16[l]
        return carry

    lax.fori_loop(0, BPW // 16, smem_fill, 0)

    lane = lax.iota(jnp.int32, 16)

    def ublock_new(k):
        kc = jnp.minimum(k, BPW - 1)
        blk = uidx_s[kc] >> 7
        prev = uidx_s[jnp.maximum(kc - 1, 0)] >> 7
        return (kc == 0) | (blk != prev)

    def fire_user(k, slot):
        u = uidx_s[jnp.minimum(k, BPW - 1)]
        pltpu.async_copy(
            utT_hbm.at[:, pl.ds((u >> 7) * BLK, BLK)],
            ublk_v.at[slot], usem)

    def fire_item(k, slot):
        iv = iidx_s[jnp.minimum(k, BPW - 1)]
        pltpu.async_copy(
            itT_hbm.at[:, pl.ds((iv >> 7) * BLK, BLK)],
            iblk_v.at[slot], isem)

    nf = jnp.int32(0)
    for k0 in range(PF):
        cond = ublock_new(jnp.int32(k0))
        slot = nf & (NBUF - 1)
        pl.when(cond)(lambda k0=k0, slot=slot: fire_user(k0, slot))
        nf = nf + cond.astype(jnp.int32)
    for k0 in range(PFI):
        fire_item(k0, k0 % NBUFI)

    def ring_body(t, carry):
        nf, cus = carry
        for p in range(NBUF):
            k = t * NBUF + p
            cond_f = (k + PF < BPW) & ublock_new(k + PF)
            fslot = nf & (NBUF - 1)
            pl.when(cond_f)(
                lambda k=k, fslot=fslot: fire_user(k + PF, fslot))
            nf = nf + cond_f.astype(jnp.int32)
            islot_f = lax.rem(jnp.int32(k + PFI), jnp.int32(NBUFI))
            pl.when(k + PFI < BPW)(
                lambda k=k, islot_f=islot_f: fire_item(k + PFI, islot_f))

            cond_w = ublock_new(k)

            @pl.when(cond_w)
            def _():
                pltpu.make_async_copy(
                    utT_hbm.at[:, pl.ds(0, BLK)], ublk_v.at[0], usem).wait()

            cus = cus + cond_w.astype(jnp.int32)
            uslot = jnp.full((16,), (cus - 1) & (NBUF - 1), jnp.int32)
            pltpu.make_async_copy(
                itT_hbm.at[:, pl.ds(0, BLK)], iblk_v.at[0], isem).wait()
            islot = jnp.full(
                (16,), lax.rem(jnp.int32(k), jnp.int32(NBUFI)), jnp.int32)

            cu = jnp.full((16,), uidx_s[k] & 127, jnp.int32)
            ci = jnp.full((16,), iidx_s[k] & 127, jnp.int32)
            kk = jnp.full((16,), k, jnp.int32)
            for s in range(4):
                fvec = s * 16 + lane
                uval = plsc.load_gather(ublk_v, [uslot, fvec, cu])
                ival = plsc.load_gather(iblk_v, [islot, fvec, ci])
                plsc.store_scatter(prodT_v, [fvec, kk], uval * ival)
        return nf, cus

    lax.fori_loop(0, BPW // NBUF, ring_body, (nf, jnp.int32(0)))

    def group_body(g, carry):
        acc = jnp.zeros((16,), jnp.float32)
        for f in range(F):
            acc = acc + prodT_v[f, pl.ds(g * 16, 16)]
        out_v[pl.ds(g * 16, 16)] = 1.0 / (1.0 + jnp.exp(-acc))
        return carry

    lax.fori_loop(0, BPW // 16, group_body, 0)

    pltpu.sync_copy(out_v, out_hbm.at[wid])


def kernel(user, item, user_table, item_table):
    perm = jnp.argsort(user)
    inv = jnp.argsort(perm)
    us = user[perm].astype(jnp.int32).reshape(NW, BPW)
    its = item[perm].astype(jnp.int32).reshape(NW, BPW)
    outs = _cf_kernel(us, its, user_table.T, item_table.T)
    return outs.reshape(B)[inv]


# speedup vs baseline: 1.0094x; 1.0094x over previous
import functools

import jax
import jax.numpy as jnp
from jax import lax
from jax.experimental import pallas as pl
from jax.experimental.pallas import tpu as pltpu
from jax.experimental.pallas import tpu_sc as plsc

B = 16384
F = 64
NC = 2
NS = 16
NW = NC * NS
BPW = B // NW
BLK = 128
NBUF = 4
PF = NBUF - 1
NBUFI = 6
PFI = NBUFI - 1

_mesh = plsc.VectorSubcoreMesh(core_axis_name="c", subcore_axis_name="s")


@functools.partial(
    pl.kernel,
    mesh=_mesh,
    out_type=jax.ShapeDtypeStruct((NW, BPW), jnp.float32),
    scratch_types=[
        pltpu.VMEM((2, BPW), jnp.int32),
        pltpu.SMEM((BPW,), jnp.int32),
        pltpu.SMEM((BPW,), jnp.int32),
        pltpu.VMEM((NBUF, F, BLK), jnp.float32),
        pltpu.VMEM((NBUFI, F, BLK), jnp.float32),
        pltpu.VMEM((F, BPW), jnp.float32),
        pltpu.VMEM((BPW,), jnp.float32),
        pltpu.SemaphoreType.DMA,
        pltpu.SemaphoreType.DMA,
    ],
    compiler_params=pltpu.CompilerParams(needs_layout_passes=False),
)
def _cf_kernel(user_hbm, item_hbm, utT_hbm, itT_hbm, out_hbm,
               idxstage_v, uidx_s, iidx_s, ublk_v, iblk_v,
               prodT_v, out_v, usem, isem):
    wid = lax.axis_index("s") * NC + lax.axis_index("c")

    pltpu.sync_copy(user_hbm.at[wid], idxstage_v.at[0])
    pltpu.sync_copy(item_hbm.at[wid], idxstage_v.at[1])

    def smem_fill(g, carry):
        u16 = idxstage_v[0, pl.ds(g * 16, 16)]
        i16 = idxstage_v[1, pl.ds(g * 16, 16)]
        for l in range(16):
            uidx_s[g * 16 + l] = u16[l]
            iidx_s[g * 16 + l] = i16[l]
        return carry

    lax.fori_loop(0, BPW // 16, smem_fill, 0)

    lane = lax.iota(jnp.int32, 16)

    def ublock_new(k):
        kc = jnp.minimum(k, BPW - 1)
        blk = uidx_s[kc] >> 7
        prev = uidx_s[jnp.maximum(kc - 1, 0)] >> 7
        return (kc == 0) | (blk != prev)

    def fire_user(k, slot):
        u = uidx_s[jnp.minimum(k, BPW - 1)]
        pltpu.async_copy(
            utT_hbm.at[:, pl.ds((u >> 7) * BLK, BLK)],
            ublk_v.at[slot], usem)

    def fire_item(k, slot):
        iv = iidx_s[jnp.minimum(k, BPW - 1)]
        pltpu.async_copy(
            itT_hbm.at[:, pl.ds((iv >> 7) * BLK, BLK)],
            iblk_v.at[slot], isem)

    nf = jnp.int32(0)
    for k0 in range(PF):
        cond = ublock_new(jnp.int32(k0))
        slot = nf & (NBUF - 1)
        pl.when(cond)(lambda k0=k0, slot=slot: fire_user(k0, slot))
        nf = nf + cond.astype(jnp.int32)
    for k0 in range(PFI):
        fire_item(k0, k0 % NBUFI)

    def ring_body(t, carry):
        nf, cus = carry
        for p in range(NBUF):
            k = t * NBUF + p
            cond_f = (k + PF < BPW) & ublock_new(k + PF)
            fslot = nf & (NBUF - 1)
            pl.when(cond_f)(
                lambda k=k, fslot=fslot: fire_user(k + PF, fslot))
            nf = nf + cond_f.astype(jnp.int32)
            islot_f = lax.rem(jnp.int32(k + PFI), jnp.int32(NBUFI))
            pl.when(k + PFI < BPW)(
                lambda k=k, islot_f=islot_f: fire_item(k + PFI, islot_f))

            cond_w = ublock_new(k)

            @pl.when(cond_w)
            def _():
                pltpu.make_async_copy(
                    utT_hbm.at[:, pl.ds(0, BLK)], ublk_v.at[0], usem).wait()

            cus = cus + cond_w.astype(jnp.int32)
            uslot = jnp.full((16,), (cus - 1) & (NBUF - 1), jnp.int32)
            pltpu.make_async_copy(
                itT_hbm.at[:, pl.ds(0, BLK)], iblk_v.at[0], isem).wait()
            islot = jnp.full(
                (16,), lax.rem(jnp.int32(k), jnp.int32(NBUFI)), jnp.int32)

            cu = jnp.full((16,), uidx_s[k] & 127, jnp.int32)
            ci = jnp.full((16,), iidx_s[k] & 127, jnp.int32)
            kk = jnp.full((16,), k, jnp.int32)
            for s in range(4):
                fvec = s * 16 + lane
                uval = plsc.load_gather(ublk_v, [uslot, fvec, cu])
                ival = plsc.load_gather(iblk_v, [islot, fvec, ci])
                plsc.store_scatter(prodT_v, [fvec, kk], uval * ival)
        return nf, cus

    lax.fori_loop(0, BPW // NBUF, ring_body, (nf, jnp.int32(0)))

    def group_body(g, carry):
        acc = jnp.zeros((16,), jnp.float32)
        for f in range(F):
            acc = acc + prodT_v[f, pl.ds(g * 16, 16)]
        out_v[pl.ds(g * 16, 16)] = 1.0 / (1.0 + jnp.exp(-acc))
        return carry

    lax.fori_loop(0, BPW // 16, group_body, 0)

    pltpu.sync_copy(out_v, out_hbm.at[wid])


def kernel(user, item, user_table, item_table):
    perm = jnp.argsort(user)
    inv = jnp.argsort(perm)
    us = user[perm].astype(jnp.int32).reshape(NW, BPW)
    its = item[perm].astype(jnp.int32).reshape(NW, BPW)
    outs = _cf_kernel(us, its, user_table.T, item_table.T)
    return outs.reshape(B)[inv]
